# Initial kernel scaffold; baseline (speedup 1.0000x reference)
import jax
import jax.numpy as jnp
from jax.experimental import pallas as pl


def kernel(x, edge_index, W1, b1, W2, b2, W3, b3, Wih, Whh, bih, bhh, L1w, L1b, L2w, L2b, L3w, L3b, L4w, L4b):
    raise NotImplementedError("write your pallas kernel here")


# speedup vs baseline: 9.0827x; 9.0827x over previous
import functools

import jax
import jax.numpy as jnp
from jax import lax
from jax.experimental import pallas as pl
from jax.experimental.pallas import tpu as pltpu
from jax.experimental.pallas import tpu_sc as plsc

N_REAL = 10000
NPAD = 10240
NC, NS = 2, 16
EP = 321536


def _mesh():
    return plsc.VectorSubcoreMesh(
        core_axis_name="c", subcore_axis_name="s",
        num_cores=NC, num_subcores=NS)


def _degree_pass(dst, ones, zeros):
    E = dst.shape[0]
    ept = E // (NC * NS)
    KD = ones.shape[0]
    nch = ept // KD
    rpt = NPAD // NS

    @functools.partial(
        pl.kernel,
        out_type=jax.ShapeDtypeStruct((NC * NPAD, 16), jnp.float32),
        mesh=_mesh(),
        scratch_types=[
            pltpu.VMEM((KD,), jnp.int32),
            pltpu.VMEM((KD, 16), jnp.float32),
            pltpu.VMEM_SHARED((NPAD, 16), jnp.float32),
        ],
        compiler_params=pltpu.CompilerParams(use_tc_tiling_on_sc=False),
        name="sc_degree",
    )
    def k(dst_hbm, ones_hbm, zeros_hbm, out_hbm, idxb, onesb, hist):
        c = lax.axis_index("c")
        s = lax.axis_index("s")
        pltpu.sync_copy(ones_hbm, onesb)
        pltpu.sync_copy(zeros_hbm.at[pl.ds(s * rpt, rpt)],
                        hist.at[pl.ds(s * rpt, rpt)])
        plsc.subcore_barrier()
        tbase = (c * NS + s) * ept

        def chunk(i, carry):
            pltpu.sync_copy(dst_hbm.at[pl.ds(tbase + i * KD, KD)], idxb)
            pltpu.sync_copy(onesb, hist.at[idxb], add=True)
            return carry

        lax.fori_loop(0, nch, chunk, 0)
        plsc.subcore_barrier()
        pltpu.sync_copy(hist.at[pl.ds(s * rpt, rpt)],
                        out_hbm.at[pl.ds(c * NPAD + s * rpt, rpt)])

    return k(dst, ones, zeros)


def _edge_pass(src2, dst, table, zeros, Bc, K, label):
    ept = EP // NS
    nch = ept // K
    rpt = NPAD // NS

    @functools.partial(
        pl.kernel,
        out_type=jax.ShapeDtypeStruct((NC * NPAD, Bc), jnp.float32),
        mesh=_mesh(),
        scratch_types=[
            pltpu.VMEM((K,), jnp.int32),
            pltpu.VMEM((K,), jnp.int32),
            pltpu.VMEM((K, Bc), jnp.float32),
            pltpu.VMEM_SHARED((NPAD, Bc), jnp.float32),
            pltpu.SemaphoreType.DMA,
        ],
        compiler_params=pltpu.CompilerParams(use_tc_tiling_on_sc=False),
        name=label,
    )
    def k(src_hbm, dst_hbm, tab_hbm, zeros_hbm, out_hbm,
          srcb, dstb, rows, acc, sem):
        c = lax.axis_index("c")
        s = lax.axis_index("s")
        pltpu.sync_copy(zeros_hbm.at[pl.ds(s * rpt, rpt)],
                        acc.at[pl.ds(s * rpt, rpt)])
        plsc.subcore_barrier()
        sbase = c * EP + s * ept
        dbase = s * ept

        def chunk(i, carry):
            pltpu.sync_copy(src_hbm.at[pl.ds(sbase + i * K, K)], srcb)
            pltpu.sync_copy(dst_hbm.at[pl.ds(dbase + i * K, K)], dstb)
            pltpu.async_copy(tab_hbm.at[srcb], rows, sem).wait()
            pltpu.sync_copy(rows, acc.at[dstb], add=True)
            return carry

        lax.fori_loop(0, nch, chunk, 0)
        plsc.subcore_barrier()
        pltpu.sync_copy(acc.at[pl.ds(s * rpt, rpt)],
                        out_hbm.at[pl.ds(c * NPAD + s * rpt, rpt)])

    return k(src2, dst, table, zeros)


def _dinv_block(hist_blk):
    deg = hist_blk[0, :, 0] + hist_blk[1, :, 0] + 1.0
    return lax.rsqrt(deg)[:, None]


def _keep_mask(i, br):
    row = lax.broadcasted_iota(jnp.int32, (br, 1), 0) + i * br
    return row < N_REAL


BR = 512


def _mm_scale(x, w, hist, label):
    fo = w.shape[1]
    bc = fo // 2

    def body(x_ref, w_ref, hist_ref, out_ref):
        dinv = _dinv_block(hist_ref)
        xw = jnp.dot(x_ref[...], w_ref[...],
                     preferred_element_type=jnp.float32,
                     precision=lax.Precision.HIGHEST)
        s = jnp.where(_keep_mask(pl.program_id(0), BR), xw * dinv, 0.0)
        out_ref[0] = s[:, :bc]
        out_ref[1] = s[:, bc:]

    return pl.pallas_call(
        body,
        grid=(NPAD // BR,),
        in_specs=[
            pl.BlockSpec((BR, x.shape[1]), lambda i: (i, 0)),
            pl.BlockSpec(w.shape, lambda i: (0, 0)),
            pl.BlockSpec((2, BR, 16), lambda i: (0, i, 0)),
        ],
        out_specs=pl.BlockSpec((2, BR, bc), lambda i: (0, i, 0)),
        out_shape=jax.ShapeDtypeStruct((2, NPAD, bc), jnp.float32),
        name=label,
    )(x, w, hist)


def _combine_mm_scale(acc, s, hist, b, w, relu, label):
    fi = 2 * acc.shape[2]
    fo = w.shape[1]
    bc = fo // 2

    def body(acc_ref, s_ref, hist_ref, b_ref, w_ref, out_ref):
        dinv = _dinv_block(hist_ref)
        z = jnp.concatenate(
            [dinv * (acc_ref[0] + s_ref[0]),
             dinv * (acc_ref[1] + s_ref[1])], axis=1) + b_ref[0]
        if relu:
            z = jnp.maximum(z, 0.0)
        if fi > 128:
            zw = jnp.dot(z.astype(jnp.bfloat16),
                         w_ref[...].astype(jnp.bfloat16),
                         preferred_element_type=jnp.float32)
        else:
            zw = jnp.dot(z, w_ref[...], preferred_element_type=jnp.float32,
                         precision=lax.Precision.HIGHEST)
        sn = jnp.where(_keep_mask(pl.program_id(0), BR), zw * dinv, 0.0)
        out_ref[0] = sn[:, :bc]
        out_ref[1] = sn[:, bc:]

    return pl.pallas_call(
        body,
        grid=(NPAD // BR,),
        in_specs=[
            pl.BlockSpec((2, BR, fi // 2), lambda i: (0, i, 0)),
            pl.BlockSpec((2, BR, fi // 2), lambda i: (0, i, 0)),
            pl.BlockSpec((2, BR, 16), lambda i: (0, i, 0)),
            pl.BlockSpec((1, fi), lambda i: (0, 0)),
            pl.BlockSpec(w.shape, lambda i: (0, 0)),
        ],
        out_specs=pl.BlockSpec((2, BR, bc), lambda i: (0, i, 0)),
        out_shape=jax.ShapeDtypeStruct((2, NPAD, bc), jnp.float32),
        name=label,
    )(acc, s, hist, b, w)


def _readout(acc, s, hist, b3, bih, bhh, L1w, L1b, L2w, L2b, L3w, L3b,
             L4w, L4b):
    def body(acc_ref, s_ref, hist_ref, b3_ref, bih_ref, bhh_ref,
             l1w_ref, l1b_ref, l2w_ref, l2b_ref, l3w_ref, l3b_ref,
             l4w_ref, l4b_ref, out_ref):
        deg = hist_ref[0, :, 0] + hist_ref[1, :, 0] + 1.0
        dinv = lax.rsqrt(deg)[:, None]
        h3 = jnp.concatenate(
            [dinv * (acc_ref[0] + s_ref[0]),
             dinv * (acc_ref[1] + s_ref[1])], axis=1) + b3_ref[0]
        gates = bih_ref[...] + bhh_ref[...]
        ci = jax.nn.sigmoid(gates[:, 0:32]) * jnp.tanh(gates[:, 64:96])
        q = jax.nn.sigmoid(gates[:, 96:128]) * jnp.tanh(ci)
        e = jnp.sum(h3 * q, axis=1, keepdims=True)
        row = lax.broadcasted_iota(jnp.int32, (NPAD, 1), 0)
        e = jnp.where(row < N_REAL, e, -1e30)
        a = jnp.exp(e - jnp.max(e))
        r = jnp.sum(a * h3, axis=0, keepdims=True) / jnp.sum(a)
        qs = jnp.concatenate([q, r], axis=1)
        v = jnp.maximum(jnp.dot(qs, l1w_ref[...],
                                preferred_element_type=jnp.float32,
                     precision=lax.Precision.HIGHEST)
                        + l1b_ref[...], 0.0)
        v = jnp.dot(v, l2w_ref[...],
                    preferred_element_type=jnp.float32,
                     precision=lax.Precision.HIGHEST) + l2b_ref[...]
        v = jnp.dot(v, l3w_ref[...],
                    preferred_element_type=jnp.float32,
                     precision=lax.Precision.HIGHEST) + l3b_ref[...]
        v = jnp.dot(v, l4w_ref[...],
                    preferred_element_type=jnp.float32,
                     precision=lax.Precision.HIGHEST) + l4b_ref[...]
        out_ref[...] = v

    return pl.pallas_call(
        body,
        out_shape=jax.ShapeDtypeStruct((1, 1), jnp.float32),
        name="tc_readout",
    )(acc, s, hist, b3, bih, bhh, L1w, L1b, L2w, L2b, L3w, L3b, L4w, L4b)


def kernel(x, edge_index, W1, b1, W2, b2, W3, b3, Wih, Whh, bih, bhh,
           L1w, L1b, L2w, L2b, L3w, L3b, L4w, L4b):
    n = x.shape[0]
    e = edge_index.shape[1]
    xp = jnp.pad(x, ((0, NPAD - n), (0, 0)))
    src = edge_index[0]
    dst = edge_index[1]

    ones = jnp.ones((80, 16), jnp.float32)
    hist = _degree_pass(dst, ones, jnp.zeros((NPAD, 16), jnp.float32))
    hist = hist.reshape(2, NPAD, 16)

    padn = EP - e
    fill = jnp.full((padn,), NPAD - 1, jnp.int32)
    srcp = jnp.concatenate([src, fill])
    dstp = jnp.concatenate([dst, fill])
    src2 = jnp.concatenate([srcp, srcp + NPAD])

    s1 = _mm_scale(xp, W1, hist, "tc_mm1")
    acc1 = _edge_pass(src2, dstp, s1.reshape(2 * NPAD, 128),
                      jnp.zeros((NPAD, 128), jnp.float32),
                      128, 128, "sc_edge1").reshape(2, NPAD, 128)
    s2 = _combine_mm_scale(acc1, s1, hist, b1.reshape(1, -1), W2,
                           True, "tc_mm2")
    acc2 = _edge_pass(src2, dstp, s2.reshape(2 * NPAD, 64),
                      jnp.zeros((NPAD, 64), jnp.float32),
                      64, 128, "sc_edge2").reshape(2, NPAD, 64)
    s3 = _combine_mm_scale(acc2, s2, hist, b2.reshape(1, -1), W3,
                           False, "tc_mm3")
    acc3 = _edge_pass(src2, dstp, s3.reshape(2 * NPAD, 16),
                      jnp.zeros((NPAD, 16), jnp.float32),
                      16, 128, "sc_edge3").reshape(2, NPAD, 16)
    out = _readout(acc3, s3, hist, b3.reshape(1, -1),
                   bih.reshape(1, -1), bhh.reshape(1, -1),
                   L1w, L1b.reshape(1, -1), L2w, L2b.reshape(1, -1),
                   L3w, L3b.reshape(1, -1), L4w, L4b.reshape(1, -1))
    return out.reshape(-1)


# speedup vs baseline: 9.6280x; 1.0600x over previous
import functools

import jax
import jax.numpy as jnp
from jax import lax
from jax.experimental import pallas as pl
from jax.experimental.pallas import tpu as pltpu
from jax.experimental.pallas import tpu_sc as plsc

N_REAL = 10000
NPAD = 10240
NC, NS = 2, 16
G = 16
EPC = 327680
EPH = 163840


def _mesh():
    return plsc.VectorSubcoreMesh(
        core_axis_name="c", subcore_axis_name="s",
        num_cores=NC, num_subcores=NS)


def _degree_pass(dst, ones, zeros):
    E = dst.shape[0]
    ept = E // (NC * NS)
    KD = ones.shape[0]
    nch = ept // KD
    rpt = NPAD // NS

    @functools.partial(
        pl.kernel,
        out_type=jax.ShapeDtypeStruct((NC * NPAD, 16), jnp.float32),
        mesh=_mesh(),
        scratch_types=[
            pltpu.VMEM((KD,), jnp.int32),
            pltpu.VMEM((KD, 16), jnp.float32),
            pltpu.VMEM_SHARED((NPAD, 16), jnp.float32),
        ],
        compiler_params=pltpu.CompilerParams(use_tc_tiling_on_sc=False),
        name="sc_degree",
    )
    def k(dst_hbm, ones_hbm, zeros_hbm, out_hbm, idxb, onesb, hist):
        c = lax.axis_index("c")
        s = lax.axis_index("s")
        pltpu.sync_copy(ones_hbm, onesb)
        pltpu.sync_copy(zeros_hbm.at[pl.ds(s * rpt, rpt)],
                        hist.at[pl.ds(s * rpt, rpt)])
        plsc.subcore_barrier()
        tbase = (c * NS + s) * ept

        def chunk(i, carry):
            pltpu.sync_copy(dst_hbm.at[pl.ds(tbase + i * KD, KD)], idxb)
            pltpu.sync_copy(onesb, hist.at[idxb], add=True)
            return carry

        lax.fori_loop(0, nch, chunk, 0)
        plsc.subcore_barrier()
        pltpu.sync_copy(hist.at[pl.ds(s * rpt, rpt)],
                        out_hbm.at[pl.ds(c * NPAD + s * rpt, rpt)])

    return k(dst, ones, zeros)


def _edge_pass(src2, dst2, table, zeros, Bc, epc, label):
    ept = epc // NS
    nblk = ept // (G * 128)
    rpt = NPAD // NS

    @functools.partial(
        pl.kernel,
        out_type=jax.ShapeDtypeStruct((NC * NPAD, Bc), jnp.float32),
        mesh=_mesh(),
        scratch_types=[
            pltpu.VMEM((G, 128), jnp.int32),
            pltpu.VMEM((G, 128), jnp.int32),
            pltpu.VMEM((128, Bc), jnp.float32),
            pltpu.VMEM((128, Bc), jnp.float32),
            pltpu.VMEM_SHARED((NPAD, Bc), jnp.float32),
            pltpu.SemaphoreType.DMA,
            pltpu.SemaphoreType.DMA,
        ],
        compiler_params=pltpu.CompilerParams(use_tc_tiling_on_sc=False),
        name=label,
    )
    def k(src_hbm, dst_hbm, tab_hbm, zeros_hbm, out_hbm,
          srcb, dstb, rows0, rows1, acc, sem0, sem1):
        c = lax.axis_index("c")
        s = lax.axis_index("s")
        pltpu.sync_copy(zeros_hbm.at[pl.ds(s * rpt, rpt)],
                        acc.at[pl.ds(s * rpt, rpt)])
        plsc.subcore_barrier()
        ibase = (c * epc + s * ept) // 128
        rows = (rows0, rows1)
        sems = (sem0, sem1)

        def blk(i, carry):
            pltpu.sync_copy(src_hbm.at[pl.ds(ibase + i * G, G)], srcb)
            pltpu.sync_copy(dst_hbm.at[pl.ds(ibase + i * G, G)], dstb)
            cps = [pltpu.async_copy(tab_hbm.at[srcb.at[0]], rows[0],
                                    sems[0]), None]
            for j in range(1, G):
                cur, prv = j % 2, (j - 1) % 2
                cps[cur] = pltpu.async_copy(tab_hbm.at[srcb.at[j]],
                                            rows[cur], sems[cur])
                cps[prv].wait()
                pltpu.sync_copy(rows[prv], acc.at[dstb.at[j - 1]], add=True)
            last = (G - 1) % 2
            cps[last].wait()
            pltpu.sync_copy(rows[last], acc.at[dstb.at[G - 1]], add=True)
            return carry

        lax.fori_loop(0, nblk, blk, 0)
        plsc.subcore_barrier()
        pltpu.sync_copy(acc.at[pl.ds(s * rpt, rpt)],
                        out_hbm.at[pl.ds(c * NPAD + s * rpt, rpt)])

    return k(src2, dst2, table, zeros)


def _dinv_block(hist_blk):
    deg = hist_blk[0, :, 0] + hist_blk[1, :, 0] + 1.0
    return lax.rsqrt(deg)[:, None]


def _keep_mask(i, br):
    row = lax.broadcasted_iota(jnp.int32, (br, 1), 0) + i * br
    return row < N_REAL


BR = 512


def _dot(a, w):
    if a.shape[1] > 128:
        return jnp.dot(a.astype(jnp.bfloat16), w.astype(jnp.bfloat16),
                       preferred_element_type=jnp.float32)
    return jnp.dot(a, w, preferred_element_type=jnp.float32,
                   precision=lax.Precision.HIGHEST)


def _mm_scale(x, w, hist, split, label):
    fo = w.shape[1]
    bc = fo // 2

    def body(x_ref, w_ref, hist_ref, out_ref):
        dinv = _dinv_block(hist_ref)
        xw = _dot(x_ref[...], w_ref[...])
        s = jnp.where(_keep_mask(pl.program_id(0), BR), xw * dinv, 0.0)
        if split:
            out_ref[0] = s[:, :bc]
            out_ref[1] = s[:, bc:]
        else:
            out_ref[...] = s

    oshape = (2, NPAD, bc) if split else (NPAD, fo)
    ospec = (pl.BlockSpec((2, BR, bc), lambda i: (0, i, 0)) if split
             else pl.BlockSpec((BR, fo), lambda i: (i, 0)))
    return pl.pallas_call(
        body,
        grid=(NPAD // BR,),
        in_specs=[
            pl.BlockSpec((BR, x.shape[1]), lambda i: (i, 0)),
            pl.BlockSpec(w.shape, lambda i: (0, 0)),
            pl.BlockSpec((2, BR, 16), lambda i: (0, i, 0)),
        ],
        out_specs=ospec,
        out_shape=jax.ShapeDtypeStruct(oshape, jnp.float32),
        name=label,
    )(x, w, hist)


def _combine_mm_scale(acc, s, hist, b, w, relu, panels, label):
    fi = w.shape[0]
    fo = w.shape[1]

    def body(acc_ref, s_ref, hist_ref, b_ref, w_ref, out_ref):
        dinv = _dinv_block(hist_ref)
        if panels:
            z = jnp.concatenate(
                [dinv * (acc_ref[0] + s_ref[0]),
                 dinv * (acc_ref[1] + s_ref[1])], axis=1) + b_ref[0]
        else:
            z = dinv * (acc_ref[0] + acc_ref[1] + s_ref[...]) + b_ref[0]
        if relu:
            z = jnp.maximum(z, 0.0)
        zw = _dot(z, w_ref[...])
        sn = jnp.where(_keep_mask(pl.program_id(0), BR), zw * dinv, 0.0)
        out_ref[...] = sn

    sspec = (pl.BlockSpec((2, BR, fi // 2), lambda i: (0, i, 0)) if panels
             else pl.BlockSpec((BR, fi), lambda i: (i, 0)))
    aw = fi // 2 if panels else fi
    return pl.pallas_call(
        body,
        grid=(NPAD // BR,),
        in_specs=[
            pl.BlockSpec((2, BR, aw), lambda i: (0, i, 0)),
            sspec,
            pl.BlockSpec((2, BR, 16), lambda i: (0, i, 0)),
            pl.BlockSpec((1, fi), lambda i: (0, 0)),
            pl.BlockSpec(w.shape, lambda i: (0, 0)),
        ],
        out_specs=pl.BlockSpec((BR, fo), lambda i: (i, 0)),
        out_shape=jax.ShapeDtypeStruct((NPAD, fo), jnp.float32),
        name=label,
    )(acc, s, hist, b, w)


def _readout(acc, s, hist, b3, bih, bhh, L1w, L1b, L2w, L2b, L3w, L3b,
             L4w, L4b):
    def body(acc_ref, s_ref, hist_ref, b3_ref, bih_ref, bhh_ref,
             l1w_ref, l1b_ref, l2w_ref, l2b_ref, l3w_ref, l3b_ref,
             l4w_ref, l4b_ref, out_ref):
        deg = hist_ref[0, :, 0] + hist_ref[1, :, 0] + 1.0
        dinv = lax.rsqrt(deg)[:, None]
        h3 = dinv * (acc_ref[0] + acc_ref[1] + s_ref[...]) + b3_ref[0]
        gates = bih_ref[...] + bhh_ref[...]
        ci = jax.nn.sigmoid(gates[:, 0:32]) * jnp.tanh(gates[:, 64:96])
        q = jax.nn.sigmoid(gates[:, 96:128]) * jnp.tanh(ci)
        e = jnp.sum(h3 * q, axis=1, keepdims=True)
        row = lax.broadcasted_iota(jnp.int32, (NPAD, 1), 0)
        e = jnp.where(row < N_REAL, e, -1e30)
        a = jnp.exp(e - jnp.max(e))
        r = jnp.sum(a * h3, axis=0, keepdims=True) / jnp.sum(a)
        qs = jnp.concatenate([q, r], axis=1)
        v = jnp.maximum(_dot(qs, l1w_ref[...]) + l1b_ref[...], 0.0)
        v = _dot(v, l2w_ref[...]) + l2b_ref[...]
        v = _dot(v, l3w_ref[...]) + l3b_ref[...]
        v = _dot(v, l4w_ref[...]) + l4b_ref[...]
        out_ref[...] = v

    return pl.pallas_call(
        body,
        out_shape=jax.ShapeDtypeStruct((1, 1), jnp.float32),
        name="tc_readout",
    )(acc, s, hist, b3, bih, bhh, L1w, L1b, L2w, L2b, L3w, L3b, L4w, L4b)


def kernel(x, edge_index, W1, b1, W2, b2, W3, b3, Wih, Whh, bih, bhh,
           L1w, L1b, L2w, L2b, L3w, L3b, L4w, L4b):
    n = x.shape[0]
    e = edge_index.shape[1]
    eh = e // 2
    xp = jnp.pad(x, ((0, NPAD - n), (0, 0)))
    src = edge_index[0]
    dst = edge_index[1]

    ones = jnp.ones((80, 16), jnp.float32)
    hist = _degree_pass(dst, ones, jnp.zeros((NPAD, 16), jnp.float32))
    hist = hist.reshape(2, NPAD, 16)

    fill_c = jnp.full(((EPC - e),), NPAD - 1, jnp.int32)
    fill_h = jnp.full(((EPH - eh),), NPAD - 1, jnp.int32)
    srcp = jnp.concatenate([src, fill_c])
    dstp = jnp.concatenate([dst, fill_c])
    src_cs = jnp.concatenate([srcp, srcp + NPAD]).reshape(-1, 128)
    dst_cs = jnp.concatenate([dstp, dstp]).reshape(-1, 128)
    src_es = jnp.concatenate(
        [src[:eh], fill_h, src[eh:], fill_h]).reshape(-1, 128)
    dst_es = jnp.concatenate(
        [dst[:eh], fill_h, dst[eh:], fill_h]).reshape(-1, 128)

    z128 = jnp.zeros((NPAD, 128), jnp.float32)

    s1 = _mm_scale(xp, W1, hist, True, "tc_mm1")
    acc1 = _edge_pass(src_cs, dst_cs, s1.reshape(2 * NPAD, 128),
                      z128, 128, EPC, "sc_edge1").reshape(2, NPAD, 128)
    s2 = _combine_mm_scale(acc1, s1, hist, b1.reshape(1, -1), W2,
                           True, True, "tc_mm2")
    acc2 = _edge_pass(src_es, dst_es, s2, z128, 128, EPH,
                      "sc_edge2").reshape(2, NPAD, 128)
    s3 = _combine_mm_scale(acc2, s2, hist, b2.reshape(1, -1), W3,
                           False, False, "tc_mm3")
    acc3 = _edge_pass(src_es, dst_es, s3,
                      jnp.zeros((NPAD, 32), jnp.float32), 32, EPH,
                      "sc_edge3").reshape(2, NPAD, 32)
    out = _readout(acc3, s3, hist, b3.reshape(1, -1),
                   bih.reshape(1, -1), bhh.reshape(1, -1),
                   L1w, L1b.reshape(1, -1), L2w, L2b.reshape(1, -1),
                   L3w, L3b.reshape(1, -1), L4w, L4b.reshape(1, -1))
    return out.reshape(-1)


# speedup vs baseline: 11.6591x; 1.2110x over previous
import functools

import jax
import jax.numpy as jnp
from jax import lax
from jax.experimental import pallas as pl
from jax.experimental.pallas import tpu as pltpu
from jax.experimental.pallas import tpu_sc as plsc

N_REAL = 10000
NPAD = 10240
NC, NS = 2, 16
G = 16
EPH = 163840


def _mesh():
    return plsc.VectorSubcoreMesh(
        core_axis_name="c", subcore_axis_name="s",
        num_cores=NC, num_subcores=NS)


def _degree_pass(dst, ones, zeros):
    E = dst.shape[0]
    ept = E // (NC * NS)
    KD = ones.shape[0]
    nch = ept // KD
    rpt = NPAD // NS

    @functools.partial(
        pl.kernel,
        out_type=jax.ShapeDtypeStruct((NC * NPAD, 16), jnp.float32),
        mesh=_mesh(),
        scratch_types=[
            pltpu.VMEM((KD,), jnp.int32),
            pltpu.VMEM((KD, 16), jnp.float32),
            pltpu.VMEM_SHARED((NPAD, 16), jnp.float32),
        ],
        compiler_params=pltpu.CompilerParams(use_tc_tiling_on_sc=False),
        name="sc_degree",
    )
    def k(dst_hbm, ones_hbm, zeros_hbm, out_hbm, idxb, onesb, hist):
        c = lax.axis_index("c")
        s = lax.axis_index("s")
        pltpu.sync_copy(ones_hbm, onesb)
        pltpu.sync_copy(zeros_hbm.at[pl.ds(s * rpt, rpt)],
                        hist.at[pl.ds(s * rpt, rpt)])
        plsc.subcore_barrier()
        tbase = (c * NS + s) * ept

        def chunk(i, carry):
            pltpu.sync_copy(dst_hbm.at[pl.ds(tbase + i * KD, KD)], idxb)
            pltpu.sync_copy(onesb, hist.at[idxb], add=True)
            return carry

        lax.fori_loop(0, nch, chunk, 0)
        plsc.subcore_barrier()
        pltpu.sync_copy(hist.at[pl.ds(s * rpt, rpt)],
                        out_hbm.at[pl.ds(c * NPAD + s * rpt, rpt)])

    return k(dst, ones, zeros)


def _edge_pass(src2, dst2, table, zeros, Bc, epc, label):
    ept = epc // NS
    nblk = ept // (G * 128)
    rpt = NPAD // NS

    @functools.partial(
        pl.kernel,
        out_type=jax.ShapeDtypeStruct((NC * NPAD, Bc), jnp.float32),
        mesh=_mesh(),
        scratch_types=[
            pltpu.VMEM((G, 128), jnp.int32),
            pltpu.VMEM((G, 128), jnp.int32),
            pltpu.VMEM((128, Bc), jnp.float32),
            pltpu.VMEM((128, Bc), jnp.float32),
            pltpu.VMEM_SHARED((NPAD, Bc), jnp.float32),
            pltpu.SemaphoreType.DMA,
            pltpu.SemaphoreType.DMA,
        ],
        compiler_params=pltpu.CompilerParams(use_tc_tiling_on_sc=False),
        name=label,
    )
    def k(src_hbm, dst_hbm, tab_hbm, zeros_hbm, out_hbm,
          srcb, dstb, rows0, rows1, acc, sem0, sem1):
        c = lax.axis_index("c")
        s = lax.axis_index("s")
        pltpu.sync_copy(zeros_hbm.at[pl.ds(s * rpt, rpt)],
                        acc.at[pl.ds(s * rpt, rpt)])
        plsc.subcore_barrier()
        ibase = (c * epc + s * ept) // 128
        rows = (rows0, rows1)
        sems = (sem0, sem1)

        def blk(i, carry):
            pltpu.sync_copy(src_hbm.at[pl.ds(ibase + i * G, G)], srcb)
            pltpu.sync_copy(dst_hbm.at[pl.ds(ibase + i * G, G)], dstb)
            cps = [pltpu.async_copy(tab_hbm.at[srcb.at[0]], rows[0],
                                    sems[0]), None]
            for j in range(1, G):
                cur, prv = j % 2, (j - 1) % 2
                cps[cur] = pltpu.async_copy(tab_hbm.at[srcb.at[j]],
                                            rows[cur], sems[cur])
                cps[prv].wait()
                pltpu.sync_copy(rows[prv], acc.at[dstb.at[j - 1]], add=True)
            last = (G - 1) % 2
            cps[last].wait()
            pltpu.sync_copy(rows[last], acc.at[dstb.at[G - 1]], add=True)
            return carry

        lax.fori_loop(0, nblk, blk, 0)
        plsc.subcore_barrier()
        pltpu.sync_copy(acc.at[pl.ds(s * rpt, rpt)],
                        out_hbm.at[pl.ds(c * NPAD + s * rpt, rpt)])

    return k(src2, dst2, table, zeros)


def _dinv_block(hist_blk):
    deg = hist_blk[0, :, 0] + hist_blk[1, :, 0] + 1.0
    return lax.rsqrt(deg)[:, None]


def _keep_mask(i, br):
    row = lax.broadcasted_iota(jnp.int32, (br, 1), 0) + i * br
    return row < N_REAL


BR = 512


def _dot(a, w):
    if a.shape[1] > 128:
        return jnp.dot(a.astype(jnp.bfloat16), w.astype(jnp.bfloat16),
                       preferred_element_type=jnp.float32)
    return jnp.dot(a, w, preferred_element_type=jnp.float32,
                   precision=lax.Precision.HIGHEST)


def _scale_x(x, hist):
    def body(x_ref, hist_ref, out_ref):
        out_ref[...] = x_ref[...] * _dinv_block(hist_ref)

    return pl.pallas_call(
        body,
        grid=(NPAD // BR,),
        in_specs=[
            pl.BlockSpec((BR, 128), lambda i: (i, 0)),
            pl.BlockSpec((2, BR, 16), lambda i: (0, i, 0)),
        ],
        out_specs=pl.BlockSpec((BR, 128), lambda i: (i, 0)),
        out_shape=jax.ShapeDtypeStruct((NPAD, 128), jnp.float32),
        name="tc_scale",
    )(x, hist)


def _l1_combine(acc, t, hist, b, w1, w2):
    def body(acc_ref, t_ref, hist_ref, b_ref, w1_ref, w2_ref, out_ref):
        dinv = _dinv_block(hist_ref)
        u = dinv * (acc_ref[0] + acc_ref[1] + t_ref[...])
        z = jnp.maximum(_dot(u, w1_ref[...]) + b_ref[0], 0.0)
        s = _dot(z, w2_ref[...]) * dinv
        out_ref[...] = jnp.where(_keep_mask(pl.program_id(0), BR), s, 0.0)

    return pl.pallas_call(
        body,
        grid=(NPAD // BR,),
        in_specs=[
            pl.BlockSpec((2, BR, 128), lambda i: (0, i, 0)),
            pl.BlockSpec((BR, 128), lambda i: (i, 0)),
            pl.BlockSpec((2, BR, 16), lambda i: (0, i, 0)),
            pl.BlockSpec((1, 256), lambda i: (0, 0)),
            pl.BlockSpec((128, 256), lambda i: (0, 0)),
            pl.BlockSpec((256, 128), lambda i: (0, 0)),
        ],
        out_specs=pl.BlockSpec((BR, 128), lambda i: (i, 0)),
        out_shape=jax.ShapeDtypeStruct((NPAD, 128), jnp.float32),
        name="tc_l1",
    )(acc, t, hist, b, w1, w2)


def _combine_mm_scale(acc, s, hist, b, w, relu, label):
    fi = w.shape[0]
    fo = w.shape[1]

    def body(acc_ref, s_ref, hist_ref, b_ref, w_ref, out_ref):
        dinv = _dinv_block(hist_ref)
        z = dinv * (acc_ref[0] + acc_ref[1] + s_ref[...]) + b_ref[0]
        if relu:
            z = jnp.maximum(z, 0.0)
        zw = _dot(z, w_ref[...])
        sn = jnp.where(_keep_mask(pl.program_id(0), BR), zw * dinv, 0.0)
        out_ref[...] = sn

    return pl.pallas_call(
        body,
        grid=(NPAD // BR,),
        in_specs=[
            pl.BlockSpec((2, BR, fi), lambda i: (0, i, 0)),
            pl.BlockSpec((BR, fi), lambda i: (i, 0)),
            pl.BlockSpec((2, BR, 16), lambda i: (0, i, 0)),
            pl.BlockSpec((1, fi), lambda i: (0, 0)),
            pl.BlockSpec(w.shape, lambda i: (0, 0)),
        ],
        out_specs=pl.BlockSpec((BR, fo), lambda i: (i, 0)),
        out_shape=jax.ShapeDtypeStruct((NPAD, fo), jnp.float32),
        name=label,
    )(acc, s, hist, b, w)


def _readout(acc, s, hist, b3, bih, bhh, L1w, L1b, L2w, L2b, L3w, L3b,
             L4w, L4b):
    def body(acc_ref, s_ref, hist_ref, b3_ref, bih_ref, bhh_ref,
             l1w_ref, l1b_ref, l2w_ref, l2b_ref, l3w_ref, l3b_ref,
             l4w_ref, l4b_ref, out_ref):
        deg = hist_ref[0, :, 0] + hist_ref[1, :, 0] + 1.0
        dinv = lax.rsqrt(deg)[:, None]
        h3 = dinv * (acc_ref[0] + acc_ref[1] + s_ref[...]) + b3_ref[0]
        gates = bih_ref[...] + bhh_ref[...]
        ci = jax.nn.sigmoid(gates[:, 0:32]) * jnp.tanh(gates[:, 64:96])
        q = jax.nn.sigmoid(gates[:, 96:128]) * jnp.tanh(ci)
        e = jnp.sum(h3 * q, axis=1, keepdims=True)
        row = lax.broadcasted_iota(jnp.int32, (NPAD, 1), 0)
        e = jnp.where(row < N_REAL, e, -1e30)
        a = jnp.exp(e - jnp.max(e))
        r = jnp.sum(a * h3, axis=0, keepdims=True) / jnp.sum(a)
        qs = jnp.concatenate([q, r], axis=1)
        v = jnp.maximum(_dot(qs, l1w_ref[...]) + l1b_ref[...], 0.0)
        v = _dot(v, l2w_ref[...]) + l2b_ref[...]
        v = _dot(v, l3w_ref[...]) + l3b_ref[...]
        v = _dot(v, l4w_ref[...]) + l4b_ref[...]
        out_ref[...] = v

    return pl.pallas_call(
        body,
        out_shape=jax.ShapeDtypeStruct((1, 1), jnp.float32),
        name="tc_readout",
    )(acc, s, hist, b3, bih, bhh, L1w, L1b, L2w, L2b, L3w, L3b, L4w, L4b)


def kernel(x, edge_index, W1, b1, W2, b2, W3, b3, Wih, Whh, bih, bhh,
           L1w, L1b, L2w, L2b, L3w, L3b, L4w, L4b):
    n = x.shape[0]
    e = edge_index.shape[1]
    eh = e // 2
    xp = jnp.pad(x, ((0, NPAD - n), (0, 0)))
    src = edge_index[0]
    dst = edge_index[1]

    ones = jnp.ones((80, 16), jnp.float32)
    hist = _degree_pass(dst, ones, jnp.zeros((NPAD, 16), jnp.float32))
    hist = hist.reshape(2, NPAD, 16)

    fill_h = jnp.full(((EPH - eh),), NPAD - 1, jnp.int32)
    src_es = jnp.concatenate(
        [src[:eh], fill_h, src[eh:], fill_h]).reshape(-1, 128)
    dst_es = jnp.concatenate(
        [dst[:eh], fill_h, dst[eh:], fill_h]).reshape(-1, 128)

    z128 = jnp.zeros((NPAD, 128), jnp.float32)

    t1 = _scale_x(xp, hist)
    acc1 = _edge_pass(src_es, dst_es, t1, z128, 128, EPH,
                      "sc_edge1").reshape(2, NPAD, 128)
    s2 = _l1_combine(acc1, t1, hist, b1.reshape(1, -1),
                     W1, W2)
    acc2 = _edge_pass(src_es, dst_es, s2, z128, 128, EPH,
                      "sc_edge2").reshape(2, NPAD, 128)
    s3 = _combine_mm_scale(acc2, s2, hist, b2.reshape(1, -1), W3,
                           False, "tc_mm3")
    acc3 = _edge_pass(src_es, dst_es, s3,
                      jnp.zeros((NPAD, 32), jnp.float32), 32, EPH,
                      "sc_edge3").reshape(2, NPAD, 32)
    out = _readout(acc3, s3, hist, b3.reshape(1, -1),
                   bih.reshape(1, -1), bhh.reshape(1, -1),
                   L1w, L1b.reshape(1, -1), L2w, L2b.reshape(1, -1),
                   L3w, L3b.reshape(1, -1), L4w, L4b.reshape(1, -1))
    return out.reshape(-1)


# speedup vs baseline: 11.6994x; 1.0034x over previous
import functools

import jax
import jax.numpy as jnp
from jax import lax
from jax.experimental import pallas as pl
from jax.experimental.pallas import tpu as pltpu
from jax.experimental.pallas import tpu_sc as plsc

N_REAL = 10000
NPAD = 10240
NC, NS = 2, 16
G = 16
EPH = 163840


def _mesh():
    return plsc.VectorSubcoreMesh(
        core_axis_name="c", subcore_axis_name="s",
        num_cores=NC, num_subcores=NS)


def _degree_pass(dst, ones, zeros):
    E = dst.shape[0]
    ept = E // (NC * NS)
    KD = ones.shape[0]
    nch = ept // KD
    rpt = NPAD // NS

    @functools.partial(
        pl.kernel,
        out_type=jax.ShapeDtypeStruct((NC * NPAD, 16), jnp.float32),
        mesh=_mesh(),
        scratch_types=[
            pltpu.VMEM((KD,), jnp.int32),
            pltpu.VMEM((KD, 16), jnp.float32),
            pltpu.VMEM_SHARED((NPAD, 16), jnp.float32),
        ],
        compiler_params=pltpu.CompilerParams(use_tc_tiling_on_sc=False),
        name="sc_degree",
    )
    def k(dst_hbm, ones_hbm, zeros_hbm, out_hbm, idxb, onesb, hist):
        c = lax.axis_index("c")
        s = lax.axis_index("s")
        pltpu.sync_copy(ones_hbm, onesb)
        pltpu.sync_copy(zeros_hbm.at[pl.ds(s * rpt, rpt)],
                        hist.at[pl.ds(s * rpt, rpt)])
        plsc.subcore_barrier()
        tbase = (c * NS + s) * ept

        def chunk(i, carry):
            pltpu.sync_copy(dst_hbm.at[pl.ds(tbase + i * KD, KD)], idxb)
            pltpu.sync_copy(onesb, hist.at[idxb], add=True)
            return carry

        lax.fori_loop(0, nch, chunk, 0)
        plsc.subcore_barrier()
        pltpu.sync_copy(hist.at[pl.ds(s * rpt, rpt)],
                        out_hbm.at[pl.ds(c * NPAD + s * rpt, rpt)])

    return k(dst, ones, zeros)


def _edge_pass(src2, dst2, table, zeros, Bc, epc, label):
    ept = epc // NS
    nblk = ept // (G * 128)
    rpt = NPAD // NS

    NB = 2 if Bc > 64 else 4

    @functools.partial(
        pl.kernel,
        out_type=jax.ShapeDtypeStruct((NC * NPAD, Bc), jnp.float32),
        mesh=_mesh(),
        scratch_types=[
            pltpu.VMEM((G, 128), jnp.int32),
            pltpu.VMEM((G, 128), jnp.int32),
        ] + [pltpu.VMEM((128, Bc), jnp.float32) for _ in range(NB)] + [
            pltpu.VMEM_SHARED((NPAD, Bc), jnp.float32),
        ] + [pltpu.SemaphoreType.DMA for _ in range(2 * NB)],
        compiler_params=pltpu.CompilerParams(use_tc_tiling_on_sc=False),
        name=label,
    )
    def k(src_hbm, dst_hbm, tab_hbm, zeros_hbm, out_hbm,
          srcb, dstb, *rest):
        rows = rest[:NB]
        acc = rest[NB]
        gsem = rest[NB + 1:2 * NB + 1]
        ssem = rest[2 * NB + 1:]
        c = lax.axis_index("c")
        s = lax.axis_index("s")
        pltpu.sync_copy(zeros_hbm.at[pl.ds(s * rpt, rpt)],
                        acc.at[pl.ds(s * rpt, rpt)])
        plsc.subcore_barrier()
        ibase = (c * epc + s * ept) // 128

        def blk(i, carry):
            pltpu.sync_copy(src_hbm.at[pl.ds(ibase + i * G, G)], srcb)
            pltpu.sync_copy(dst_hbm.at[pl.ds(ibase + i * G, G)], dstb)
            gcp = [None] * NB
            scp = [None] * NB
            for j in range(G):
                cur = j % NB
                if j >= NB:
                    scp[cur].wait()
                gcp[cur] = pltpu.async_copy(tab_hbm.at[srcb.at[j]],
                                            rows[cur], gsem[cur])
                if j >= 1:
                    prv = (j - 1) % NB
                    gcp[prv].wait()
                    scp[prv] = pltpu.async_copy(
                        rows[prv], acc.at[dstb.at[j - 1]], ssem[prv],
                        add=True)
            last = (G - 1) % NB
            gcp[last].wait()
            scp[last] = pltpu.async_copy(
                rows[last], acc.at[dstb.at[G - 1]], ssem[last], add=True)
            for r in range(NB):
                scp[(last + 1 + r) % NB].wait()
            return carry

        lax.fori_loop(0, nblk, blk, 0)
        plsc.subcore_barrier()
        pltpu.sync_copy(acc.at[pl.ds(s * rpt, rpt)],
                        out_hbm.at[pl.ds(c * NPAD + s * rpt, rpt)])

    return k(src2, dst2, table, zeros)


def _dinv_block(hist_blk):
    deg = hist_blk[0, :, 0] + hist_blk[1, :, 0] + 1.0
    return lax.rsqrt(deg)[:, None]


def _keep_mask(i, br):
    row = lax.broadcasted_iota(jnp.int32, (br, 1), 0) + i * br
    return row < N_REAL


BR = 512


def _dot(a, w):
    if a.shape[1] > 128:
        return jnp.dot(a.astype(jnp.bfloat16), w.astype(jnp.bfloat16),
                       preferred_element_type=jnp.float32)
    return jnp.dot(a, w, preferred_element_type=jnp.float32,
                   precision=lax.Precision.HIGHEST)


def _scale_x(x, hist):
    def body(x_ref, hist_ref, out_ref):
        out_ref[...] = x_ref[...] * _dinv_block(hist_ref)

    return pl.pallas_call(
        body,
        grid=(NPAD // BR,),
        in_specs=[
            pl.BlockSpec((BR, 128), lambda i: (i, 0)),
            pl.BlockSpec((2, BR, 16), lambda i: (0, i, 0)),
        ],
        out_specs=pl.BlockSpec((BR, 128), lambda i: (i, 0)),
        out_shape=jax.ShapeDtypeStruct((NPAD, 128), jnp.float32),
        name="tc_scale",
    )(x, hist)


def _l1_combine(acc, t, hist, b, w1, w2):
    def body(acc_ref, t_ref, hist_ref, b_ref, w1_ref, w2_ref, out_ref):
        dinv = _dinv_block(hist_ref)
        u = dinv * (acc_ref[0] + acc_ref[1] + t_ref[...])
        z = jnp.maximum(_dot(u, w1_ref[...]) + b_ref[0], 0.0)
        s = _dot(z, w2_ref[...]) * dinv
        out_ref[...] = jnp.where(_keep_mask(pl.program_id(0), BR), s, 0.0)

    return pl.pallas_call(
        body,
        grid=(NPAD // BR,),
        in_specs=[
            pl.BlockSpec((2, BR, 128), lambda i: (0, i, 0)),
            pl.BlockSpec((BR, 128), lambda i: (i, 0)),
            pl.BlockSpec((2, BR, 16), lambda i: (0, i, 0)),
            pl.BlockSpec((1, 256), lambda i: (0, 0)),
            pl.BlockSpec((128, 256), lambda i: (0, 0)),
            pl.BlockSpec((256, 128), lambda i: (0, 0)),
        ],
        out_specs=pl.BlockSpec((BR, 128), lambda i: (i, 0)),
        out_shape=jax.ShapeDtypeStruct((NPAD, 128), jnp.float32),
        name="tc_l1",
    )(acc, t, hist, b, w1, w2)


def _combine_mm_scale(acc, s, hist, b, w, relu, label):
    fi = w.shape[0]
    fo = w.shape[1]

    def body(acc_ref, s_ref, hist_ref, b_ref, w_ref, out_ref):
        dinv = _dinv_block(hist_ref)
        z = dinv * (acc_ref[0] + acc_ref[1] + s_ref[...]) + b_ref[0]
        if relu:
            z = jnp.maximum(z, 0.0)
        zw = _dot(z, w_ref[...])
        sn = jnp.where(_keep_mask(pl.program_id(0), BR), zw * dinv, 0.0)
        out_ref[...] = sn

    return pl.pallas_call(
        body,
        grid=(NPAD // BR,),
        in_specs=[
            pl.BlockSpec((2, BR, fi), lambda i: (0, i, 0)),
            pl.BlockSpec((BR, fi), lambda i: (i, 0)),
            pl.BlockSpec((2, BR, 16), lambda i: (0, i, 0)),
            pl.BlockSpec((1, fi), lambda i: (0, 0)),
            pl.BlockSpec(w.shape, lambda i: (0, 0)),
        ],
        out_specs=pl.BlockSpec((BR, fo), lambda i: (i, 0)),
        out_shape=jax.ShapeDtypeStruct((NPAD, fo), jnp.float32),
        name=label,
    )(acc, s, hist, b, w)


def _readout(acc, s, hist, b3, bih, bhh, L1w, L1b, L2w, L2b, L3w, L3b,
             L4w, L4b):
    def body(acc_ref, s_ref, hist_ref, b3_ref, bih_ref, bhh_ref,
             l1w_ref, l1b_ref, l2w_ref, l2b_ref, l3w_ref, l3b_ref,
             l4w_ref, l4b_ref, out_ref):
        deg = hist_ref[0, :, 0] + hist_ref[1, :, 0] + 1.0
        dinv = lax.rsqrt(deg)[:, None]
        h3 = dinv * (acc_ref[0] + acc_ref[1] + s_ref[...]) + b3_ref[0]
        gates = bih_ref[...] + bhh_ref[...]
        ci = jax.nn.sigmoid(gates[:, 0:32]) * jnp.tanh(gates[:, 64:96])
        q = jax.nn.sigmoid(gates[:, 96:128]) * jnp.tanh(ci)
        e = jnp.sum(h3 * q, axis=1, keepdims=True)
        row = lax.broadcasted_iota(jnp.int32, (NPAD, 1), 0)
        e = jnp.where(row < N_REAL, e, -1e30)
        a = jnp.exp(e - jnp.max(e))
        r = jnp.sum(a * h3, axis=0, keepdims=True) / jnp.sum(a)
        qs = jnp.concatenate([q, r], axis=1)
        v = jnp.maximum(_dot(qs, l1w_ref[...]) + l1b_ref[...], 0.0)
        v = _dot(v, l2w_ref[...]) + l2b_ref[...]
        v = _dot(v, l3w_ref[...]) + l3b_ref[...]
        v = _dot(v, l4w_ref[...]) + l4b_ref[...]
        out_ref[...] = v

    return pl.pallas_call(
        body,
        out_shape=jax.ShapeDtypeStruct((1, 1), jnp.float32),
        name="tc_readout",
    )(acc, s, hist, b3, bih, bhh, L1w, L1b, L2w, L2b, L3w, L3b, L4w, L4b)


def kernel(x, edge_index, W1, b1, W2, b2, W3, b3, Wih, Whh, bih, bhh,
           L1w, L1b, L2w, L2b, L3w, L3b, L4w, L4b):
    n = x.shape[0]
    e = edge_index.shape[1]
    eh = e // 2
    xp = jnp.pad(x, ((0, NPAD - n), (0, 0)))
    src = edge_index[0]
    dst = edge_index[1]

    ones = jnp.ones((80, 16), jnp.float32)
    hist = _degree_pass(dst, ones, jnp.zeros((NPAD, 16), jnp.float32))
    hist = hist.reshape(2, NPAD, 16)

    fill_h = jnp.full(((EPH - eh),), NPAD - 1, jnp.int32)
    src_es = jnp.concatenate(
        [src[:eh], fill_h, src[eh:], fill_h]).reshape(-1, 128)
    dst_es = jnp.concatenate(
        [dst[:eh], fill_h, dst[eh:], fill_h]).reshape(-1, 128)

    z128 = jnp.zeros((NPAD, 128), jnp.float32)

    t1 = _scale_x(xp, hist)
    acc1 = _edge_pass(src_es, dst_es, t1, z128, 128, EPH,
                      "sc_edge1").reshape(2, NPAD, 128)
    s2 = _l1_combine(acc1, t1, hist, b1.reshape(1, -1),
                     W1, W2)
    acc2 = _edge_pass(src_es, dst_es, s2, z128, 128, EPH,
                      "sc_edge2").reshape(2, NPAD, 128)
    s3 = _combine_mm_scale(acc2, s2, hist, b2.reshape(1, -1), W3,
                           False, "tc_mm3")
    acc3 = _edge_pass(src_es, dst_es, s3,
                      jnp.zeros((NPAD, 32), jnp.float32), 32, EPH,
                      "sc_edge3").reshape(2, NPAD, 32)
    out = _readout(acc3, s3, hist, b3.reshape(1, -1),
                   bih.reshape(1, -1), bhh.reshape(1, -1),
                   L1w, L1b.reshape(1, -1), L2w, L2b.reshape(1, -1),
                   L3w, L3b.reshape(1, -1), L4w, L4b.reshape(1, -1))
    return out.reshape(-1)
